# baseline (speedup 1.0000x reference)
import jax
import jax.numpy as jnp
from jax import lax
from jax.experimental import pallas as pl

N = 10000
E = 160000
D = 256
EB = 800
NB = 400
BM = 400

_f32 = jnp.float32


def _seg_tc_body(src_ref, dst_ref, x_ref, out_ref, deg_ref):
    @pl.when(pl.program_id(0) == 0)
    def _():
        out_ref[...] = jnp.zeros_like(out_ref)
        deg_ref[...] = jnp.zeros_like(deg_ref)

    srcv = src_ref[0, 0]
    dstv = dst_ref[0, 0]
    iota = lax.broadcasted_iota(jnp.int32, (EB, NB), 1)

    msg = jnp.zeros((EB, D), _f32)
    for sb in range(N // NB):
        oh = (srcv[:, None] == (sb * NB + iota)).astype(_f32)
        msg = msg + jnp.dot(oh, x_ref[pl.ds(sb * NB, NB), :],
                            preferred_element_type=_f32)

    for db in range(N // NB):
        ohd = (dstv[:, None] == (db * NB + iota)).astype(_f32)
        upd = lax.dot_general(ohd, msg, (((0,), (0,)), ((), ())),
                              preferred_element_type=_f32)
        out_ref[pl.ds(db * NB, NB), :] += upd
        dcol = jnp.sum(ohd, axis=0)
        deg_ref[pl.ds(db * NB, NB), :] += dcol[:, None]


_seg_tc = pl.pallas_call(
    _seg_tc_body,
    grid=(E // EB,),
    in_specs=[
        pl.BlockSpec((1, 1, EB), lambda i: (i, 0, 0)),
        pl.BlockSpec((1, 1, EB), lambda i: (i, 0, 0)),
        pl.BlockSpec((N, D), lambda i: (0, 0)),
    ],
    out_specs=[
        pl.BlockSpec((N, D), lambda i: (0, 0)),
        pl.BlockSpec((N, 8), lambda i: (0, 0)),
    ],
    out_shape=[
        jax.ShapeDtypeStruct((N, D), _f32),
        jax.ShapeDtypeStruct((N, 8), _f32),
    ],
)


def _full(shape):
    return pl.BlockSpec(shape, lambda i: (0,) * len(shape))


def _mm1_body(s_ref, deg_ref, x_ref, wl_ref, wr_ref, b_ref, h_ref):
    deg = jnp.maximum(deg_ref[...][:, 0:1], 1.0)
    agg = s_ref[...] / deg
    h = (jnp.dot(agg, wl_ref[...], preferred_element_type=_f32)
         + jnp.dot(x_ref[...], wr_ref[...], preferred_element_type=_f32)
         + b_ref[...])
    h_ref[...] = jnp.maximum(h, 0.0)


_mm1 = pl.pallas_call(
    _mm1_body,
    grid=(N // BM,),
    in_specs=[
        pl.BlockSpec((BM, D), lambda i: (i, 0)),
        pl.BlockSpec((BM, 8), lambda i: (i, 0)),
        pl.BlockSpec((BM, D), lambda i: (i, 0)),
        _full((D, D)), _full((D, D)), _full((1, D)),
    ],
    out_specs=pl.BlockSpec((BM, D), lambda i: (i, 0)),
    out_shape=jax.ShapeDtypeStruct((N, D), _f32),
)


def _mm2_body(s_ref, deg_ref, h_ref, wl_ref, wr_ref, b_ref, wfc_ref,
              bfc_ref, o_ref):
    deg = jnp.maximum(deg_ref[...][:, 0:1], 1.0)
    agg = s_ref[...] / deg
    h2 = (jnp.dot(agg, wl_ref[...], preferred_element_type=_f32)
          + jnp.dot(h_ref[...], wr_ref[...], preferred_element_type=_f32)
          + b_ref[...])
    h2 = jnp.maximum(h2, 0.0)
    o_ref[...] = (jnp.dot(h2, wfc_ref[...], preferred_element_type=_f32)
                  + bfc_ref[...])


_mm2 = pl.pallas_call(
    _mm2_body,
    grid=(N // BM,),
    in_specs=[
        pl.BlockSpec((BM, D), lambda i: (i, 0)),
        pl.BlockSpec((BM, 8), lambda i: (i, 0)),
        pl.BlockSpec((BM, D), lambda i: (i, 0)),
        _full((D, D)), _full((D, D)), _full((1, D)),
        _full((D, D)), _full((1, D)),
    ],
    out_specs=pl.BlockSpec((BM, D), lambda i: (i, 0)),
    out_shape=jax.ShapeDtypeStruct((N, D), _f32),
)


def kernel(x, edge_index, W1l, W1r, b1, W2l, W2r, b2, Wfc, bfc):
    src = edge_index[0].astype(jnp.int32).reshape(E // EB, 1, EB)
    dst = edge_index[1].astype(jnp.int32).reshape(E // EB, 1, EB)

    s1, deg = _seg_tc(src, dst, x)
    h1 = _mm1(s1, deg, x, W1l.T, W1r.T, b1.reshape(1, D))
    s2, _ = _seg_tc(src, dst, h1)
    out = _mm2(s2, deg, h1, W2l.T, W2r.T, b2.reshape(1, D),
               Wfc.T, bfc.reshape(1, D))
    return out


# speedup vs baseline: 1.0222x; 1.0222x over previous
import jax
import jax.numpy as jnp
from jax import lax
from jax.experimental import pallas as pl

N = 10000
E = 160000
D = 256
EB = 800
NB = 400
BM = 400

_f32 = jnp.float32


def _seg_tc_body(src_ref, dst_ref, x_ref, out_ref, deg_ref):
    @pl.when(pl.program_id(0) == 0)
    def _():
        out_ref[...] = jnp.zeros_like(out_ref)
        deg_ref[...] = jnp.zeros_like(deg_ref)

    srcv = src_ref[0, 0]
    dstv = dst_ref[0, 0]
    iota = lax.broadcasted_iota(jnp.int32, (EB, NB), 1)

    bf16 = jnp.bfloat16
    msg = jnp.zeros((EB, D), _f32)
    for sb in range(N // NB):
        oh = (srcv[:, None] == (sb * NB + iota)).astype(bf16)
        msg = msg + jnp.dot(oh, x_ref[pl.ds(sb * NB, NB), :].astype(bf16),
                            preferred_element_type=_f32)

    msg16 = msg.astype(bf16)
    for db in range(N // NB):
        ohd = (dstv[:, None] == (db * NB + iota)).astype(bf16)
        upd = lax.dot_general(ohd, msg16, (((0,), (0,)), ((), ())),
                              preferred_element_type=_f32)
        out_ref[pl.ds(db * NB, NB), :] += upd
        dcol = jnp.sum(ohd.astype(_f32), axis=0)
        deg_ref[pl.ds(db * NB, NB), :] += dcol[:, None]


_seg_tc = pl.pallas_call(
    _seg_tc_body,
    grid=(E // EB,),
    in_specs=[
        pl.BlockSpec((1, 1, EB), lambda i: (i, 0, 0)),
        pl.BlockSpec((1, 1, EB), lambda i: (i, 0, 0)),
        pl.BlockSpec((N, D), lambda i: (0, 0)),
    ],
    out_specs=[
        pl.BlockSpec((N, D), lambda i: (0, 0)),
        pl.BlockSpec((N, 8), lambda i: (0, 0)),
    ],
    out_shape=[
        jax.ShapeDtypeStruct((N, D), _f32),
        jax.ShapeDtypeStruct((N, 8), _f32),
    ],
)


def _full(shape):
    return pl.BlockSpec(shape, lambda i: (0,) * len(shape))


def _mm1_body(s_ref, deg_ref, x_ref, wl_ref, wr_ref, b_ref, h_ref):
    deg = jnp.maximum(deg_ref[...][:, 0:1], 1.0)
    agg = s_ref[...] / deg
    h = (jnp.dot(agg, wl_ref[...], preferred_element_type=_f32)
         + jnp.dot(x_ref[...], wr_ref[...], preferred_element_type=_f32)
         + b_ref[...])
    h_ref[...] = jnp.maximum(h, 0.0)


_mm1 = pl.pallas_call(
    _mm1_body,
    grid=(N // BM,),
    in_specs=[
        pl.BlockSpec((BM, D), lambda i: (i, 0)),
        pl.BlockSpec((BM, 8), lambda i: (i, 0)),
        pl.BlockSpec((BM, D), lambda i: (i, 0)),
        _full((D, D)), _full((D, D)), _full((1, D)),
    ],
    out_specs=pl.BlockSpec((BM, D), lambda i: (i, 0)),
    out_shape=jax.ShapeDtypeStruct((N, D), _f32),
)


def _mm2_body(s_ref, deg_ref, h_ref, wl_ref, wr_ref, b_ref, wfc_ref,
              bfc_ref, o_ref):
    deg = jnp.maximum(deg_ref[...][:, 0:1], 1.0)
    agg = s_ref[...] / deg
    h2 = (jnp.dot(agg, wl_ref[...], preferred_element_type=_f32)
          + jnp.dot(h_ref[...], wr_ref[...], preferred_element_type=_f32)
          + b_ref[...])
    h2 = jnp.maximum(h2, 0.0)
    o_ref[...] = (jnp.dot(h2, wfc_ref[...], preferred_element_type=_f32)
                  + bfc_ref[...])


_mm2 = pl.pallas_call(
    _mm2_body,
    grid=(N // BM,),
    in_specs=[
        pl.BlockSpec((BM, D), lambda i: (i, 0)),
        pl.BlockSpec((BM, 8), lambda i: (i, 0)),
        pl.BlockSpec((BM, D), lambda i: (i, 0)),
        _full((D, D)), _full((D, D)), _full((1, D)),
        _full((D, D)), _full((1, D)),
    ],
    out_specs=pl.BlockSpec((BM, D), lambda i: (i, 0)),
    out_shape=jax.ShapeDtypeStruct((N, D), _f32),
)


def kernel(x, edge_index, W1l, W1r, b1, W2l, W2r, b2, Wfc, bfc):
    src = edge_index[0].astype(jnp.int32).reshape(E // EB, 1, EB)
    dst = edge_index[1].astype(jnp.int32).reshape(E // EB, 1, EB)

    s1, deg = _seg_tc(src, dst, x)
    h1 = _mm1(s1, deg, x, W1l.T, W1r.T, b1.reshape(1, D))
    s2, _ = _seg_tc(src, dst, h1)
    out = _mm2(s2, deg, h1, W2l.T, W2r.T, b2.reshape(1, D),
               Wfc.T, bfc.reshape(1, D))
    return out
